# baseline (speedup 1.0000x reference)
import functools

import jax
import jax.numpy as jnp
from jax import lax
from jax.experimental import pallas as pl
from jax.experimental.pallas import tpu as pltpu
from jax.experimental.pallas import tpu_sc as plsc

_SMOOTHING = 0.1
_CONFIDENCE = 1.0 - _SMOOTHING


def _row_pass_body(x_ref, o_ref, m_ref, s_ref, sx_ref, *, num_classes):
    j = pl.program_id(1)

    @pl.when(j == 0)
    def _init():
        m_ref[...] = jnp.full_like(m_ref, -1e30)
        s_ref[...] = jnp.zeros_like(s_ref)
        sx_ref[...] = jnp.zeros_like(sx_ref)

    x = x_ref[...]
    bm = jnp.max(x, axis=1, keepdims=True)
    m_old = m_ref[...]
    m_new = jnp.maximum(m_old, bm)
    s_ref[...] = (s_ref[...] * jnp.exp(m_old - m_new)
                  + jnp.sum(jnp.exp(x - m_new), axis=1, keepdims=True))
    m_ref[...] = m_new
    sx_ref[...] = sx_ref[...] + jnp.sum(x, axis=1, keepdims=True)

    @pl.when(j == pl.num_programs(1) - 1)
    def _finish():
        eps = _SMOOTHING / (num_classes - 1)
        lse = m_ref[...] + jnp.log(s_ref[...])
        o_ref[...] = (eps * (num_classes * lse - sx_ref[...])
                      + (_CONFIDENCE - eps) * lse)


def _pick_col_block(c, cap=6400):
    if c <= cap:
        return c
    best = 128
    for k in range(128, cap + 1, 128):
        if c % k == 0:
            best = k
    return best


def _tc_row_pass(outputs):
    n, c = outputs.shape
    r = 256 if n % 256 == 0 else n
    cc = _pick_col_block(c)
    return pl.pallas_call(
        functools.partial(_row_pass_body, num_classes=c),
        grid=(n // r, c // cc),
        in_specs=[pl.BlockSpec((r, cc), lambda i, j: (i, j))],
        out_specs=pl.BlockSpec((r, 1), lambda i, j: (i, 0)),
        out_shape=jax.ShapeDtypeStruct((n, 1), jnp.float32),
        scratch_shapes=[
            pltpu.VMEM((r, 1), jnp.float32),
            pltpu.VMEM((r, 1), jnp.float32),
            pltpu.VMEM((r, 1), jnp.float32),
        ],
        compiler_params=pltpu.CompilerParams(
            dimension_semantics=("parallel", "arbitrary"),
        ),
    )(outputs)


def _sc_gather_xt(outputs, targets):
    n, c = outputs.shape
    info = plsc.get_sparse_core_info()
    nc, ns, nl = info.num_cores, info.num_subcores, info.num_lanes
    nw = nc * ns
    b_per_w = n // nw
    table = outputs.reshape(n * c)
    mesh = plsc.VectorSubcoreMesh(core_axis_name="c", subcore_axis_name="s")

    @functools.partial(
        pl.kernel,
        mesh=mesh,
        out_type=jax.ShapeDtypeStruct((n,), jnp.float32),
        scratch_types=[
            pltpu.VMEM((b_per_w,), jnp.int32),
            pltpu.VMEM((b_per_w,), jnp.int32),
            pltpu.VMEM((b_per_w,), jnp.float32),
            pltpu.SemaphoreType.DMA,
        ],
    )
    def k(table_hbm, tgt_hbm, out_hbm, tgt_v, idx_v, val_v, sem):
        wid = lax.axis_index("s") * nc + lax.axis_index("c")
        base = wid * b_per_w
        pltpu.sync_copy(tgt_hbm.at[pl.ds(base, b_per_w)], tgt_v)
        for kk in range(b_per_w // nl):
            t16 = tgt_v[pl.ds(kk * nl, nl)]
            i16 = (base + kk * nl) + lax.iota(jnp.int32, nl)
            idx_v[pl.ds(kk * nl, nl)] = i16 * c + t16
        pltpu.async_copy(table_hbm.at[idx_v], val_v, sem).wait()
        pltpu.sync_copy(val_v, out_hbm.at[pl.ds(base, b_per_w)])

    return k(table, targets)


def _fin_body(a_ref, xt_ref, o_ref, *, num_classes):
    eps = _SMOOTHING / (num_classes - 1)
    n = a_ref.shape[0]
    o_ref[...] = jnp.sum(
        a_ref[...] - (_CONFIDENCE - eps) * xt_ref[...],
        keepdims=True) * (1.0 / n)


def kernel(outputs, targets):
    n, c = outputs.shape
    a = _tc_row_pass(outputs)
    xt = _sc_gather_xt(outputs, targets)
    loss = pl.pallas_call(
        functools.partial(_fin_body, num_classes=c),
        out_shape=jax.ShapeDtypeStruct((1, 1), jnp.float32),
    )(a, xt.reshape(n, 1))
    return loss[0, 0]


# speedup vs baseline: 2.7367x; 2.7367x over previous
import functools

import jax
import jax.numpy as jnp
from jax.experimental import pallas as pl
from jax.experimental.pallas import tpu as pltpu

_SMOOTHING = 0.1
_CONFIDENCE = 1.0 - _SMOOTHING


def _row_pass_body(x_ref, t_ref, o_ref, *, num_classes):
    x = x_ref[...]
    r = x.shape[0]
    c = x.shape[1]
    bm = jnp.max(x, axis=1, keepdims=True)
    s = jnp.sum(jnp.exp(x - bm), axis=1, keepdims=True)
    sx = jnp.sum(x, axis=1, keepdims=True)
    cols = jax.lax.broadcasted_iota(jnp.int32, (r, c), 1)
    xt = jnp.sum(jnp.where(cols == t_ref[...], x, 0.0), axis=1, keepdims=True)
    eps = _SMOOTHING / (num_classes - 1)
    lse = bm + jnp.log(s)
    o_ref[...] = (eps * (num_classes * lse - sx)
                  + (_CONFIDENCE - eps) * (lse - xt))


def _mean_body(r_ref, o_ref):
    n = r_ref.shape[0]
    o_ref[...] = jnp.sum(r_ref[...], keepdims=True) * (1.0 / n)


def kernel(outputs, targets):
    n, c = outputs.shape
    r = 128 if n % 128 == 0 else n
    t2 = targets.reshape(n, 1)

    row_losses = pl.pallas_call(
        functools.partial(_row_pass_body, num_classes=c),
        grid=(n // r,),
        in_specs=[
            pl.BlockSpec((r, c), lambda i: (i, 0)),
            pl.BlockSpec((r, 1), lambda i: (i, 0)),
        ],
        out_specs=pl.BlockSpec((r, 1), lambda i: (i, 0)),
        out_shape=jax.ShapeDtypeStruct((n, 1), jnp.float32),
        compiler_params=pltpu.CompilerParams(
            dimension_semantics=("arbitrary",),
        ),
    )(outputs, t2)

    loss = pl.pallas_call(
        _mean_body,
        out_shape=jax.ShapeDtypeStruct((1, 1), jnp.float32),
    )(row_losses)
    return loss[0, 0]
